# baseline (speedup 1.0000x reference)
import functools

import jax
import jax.numpy as jnp
from jax import lax
from jax.experimental import pallas as pl
from jax.experimental.pallas import tpu as pltpu
from jax.experimental.pallas import tpu_sc as plsc

B = 16384
D = 64
NC = 2
NS = 16
NW = NC * NS
BPW = B // NW
R = 16
NBL = BPW // R

_mesh = plsc.VectorSubcoreMesh(core_axis_name="c", subcore_axis_name="s")
_params = pltpu.CompilerParams(needs_layout_passes=False,
                               disable_bounds_checks=True)


@functools.partial(
    pl.kernel,
    mesh=_mesh,
    out_type=tuple(jax.ShapeDtypeStruct((B, D), jnp.float32)
                   for _ in range(4)),
    scratch_types=[
        tuple(pltpu.VMEM((BPW,), jnp.int32) for _ in range(4)),
        tuple(pltpu.SemaphoreType.DMA for _ in range(16)),
    ],
    compiler_params=_params,
)
def _gather(user_t, item_t, users_hbm, pos_hbm, pot_hbm, neg_hbm,
            out_u, out_p, out_t, out_n, ibufs, sems):
    wid = lax.axis_index("s") * NC + lax.axis_index("c")
    b0 = wid * BPW

    streams = ((users_hbm, user_t, out_u, 0),
               (pos_hbm, item_t, out_p, 1),
               (pot_hbm, item_t, out_t, 2),
               (neg_hbm, item_t, out_n, 3))

    for idx_hbm, _, _, k in streams:
        pltpu.sync_copy(idx_hbm.at[pl.ds(b0, BPW)], ibufs[k])

    for _, tab, out, k in streams:
        ib = ibufs[k]

        def wait(s, tab=tab, out=out):
            pltpu.make_async_copy(tab.at[pl.ds(0, 1)],
                                  out.at[pl.ds(b0, 1)], sems[s]).wait()

        def body(g, _, tab=tab, out=out, ib=ib):
            iv16 = ib[pl.ds(g * R, R)]

            @pl.when(g > 0)
            def _():
                for s in range(R):
                    wait(s)

            for s in range(R):
                pltpu.async_copy(tab.at[pl.ds(iv16[s], 1)],
                                 out.at[pl.ds(b0 + g * R + s, 1)], sems[s])
            return 0

        lax.fori_loop(0, NBL, body, 0)
        for s in range(R):
            wait(s)


def kernel(user_emb, item_emb, users, pos_items, pot_items, neg_items):
    return _gather(user_emb, item_emb,
                   users.astype(jnp.int32), pos_items.astype(jnp.int32),
                   pot_items.astype(jnp.int32), neg_items.astype(jnp.int32))
